# Initial kernel scaffold; baseline (speedup 1.0000x reference)
import functools

import jax
import jax.numpy as jnp
from jax import lax
from jax.experimental import pallas as pl
from jax.experimental.pallas import tpu as pltpu
from jax.experimental.pallas import tpu_sc as plsc

_DIM = 128
_MAXU = 64
_BATCH = 16384
_L = 16
_NC = 2
_NS = 16
_NW = _NC * _NS
_BPW = _BATCH // _NW
_IDX_CHUNK = 128


def _body(fl_hbm, w_hbm, out_hbm, fl_v, hist_v, rank_v, idx2_v, rows_v, sem):
    wid = lax.axis_index("s") * _NC + lax.axis_index("c")
    base = wid * _BPW

    pltpu.sync_copy(fl_hbm, fl_v)

    zeros = jnp.zeros((_L,), jnp.int32)
    for j in range(_MAXU // _L):
        hist_v[pl.ds(j * _L, _L)] = zeros

    ones = jnp.ones((_L,), jnp.int32)

    def scan_step(i, carry):
        for j in range(8):
            v = fl_v[pl.ds((i * 8 + j) * _L, _L)]
            plsc.store_scatter(hist_v, [v], ones)
        return carry

    lax.fori_loop(0, _BATCH // (_L * 8), scan_step, jnp.int32(0))

    running = jnp.int32(0)
    for j in range(_MAXU // _L):
        seg = hist_v[pl.ds(j * _L, _L)]
        inc = plsc.cumsum(seg)
        rank_v[pl.ds(j * _L, _L)] = (inc - seg) + running
        running = running + jnp.sum(seg)

    for i in range(_BPW // _L):
        v = fl_v[pl.ds(base + i * _L, _L)]
        r = plsc.load_gather(rank_v, [v])
        idx2_v[i * _L // _IDX_CHUNK, pl.ds((i * _L) % _IDX_CHUNK, _L)] = r

    copies = []
    for g in range(_BPW // _IDX_CHUNK):
        copies.append(
            pltpu.async_copy(
                w_hbm.at[idx2_v.at[g]],
                rows_v.at[pl.ds(g * _IDX_CHUNK, _IDX_CHUNK)],
                sem,
            )
        )
    for c in copies:
        c.wait()

    pltpu.sync_copy(rows_v, out_hbm.at[pl.ds(base, _BPW)])


def kernel(file_list, W):
    mesh = plsc.VectorSubcoreMesh(
        core_axis_name="c", subcore_axis_name="s", num_cores=_NC, num_subcores=_NS
    )
    run = functools.partial(
        pl.kernel,
        out_type=jax.ShapeDtypeStruct((_BATCH, _DIM), jnp.float32),
        mesh=mesh,
        scratch_types=[
            pltpu.VMEM((_BATCH,), jnp.int32),
            pltpu.VMEM((_MAXU,), jnp.int32),
            pltpu.VMEM((_MAXU,), jnp.int32),
            pltpu.VMEM((_BPW // _IDX_CHUNK, _IDX_CHUNK), jnp.int32),
            pltpu.VMEM((_BPW, _DIM), jnp.float32),
            pltpu.SemaphoreType.DMA,
        ],
    )(_body)
    return run(file_list, W)


# speedup vs baseline: 14.9092x; 14.9092x over previous
import functools

import jax
import jax.numpy as jnp
from jax import lax
from jax.experimental import pallas as pl
from jax.experimental.pallas import tpu as pltpu
from jax.experimental.pallas import tpu_sc as plsc

_DIM = 128
_MAXU = 64
_BATCH = 16384
_L = 16
_NC = 2
_NS = 16
_NW = _NC * _NS
_BPW = _BATCH // _NW
_IDX_CHUNK = 128


def _body(fl_hbm, w_hbm, out_hbm, fl_v, hist_v, rank_v, idx2_v, rows_v, sem):
    wid = lax.axis_index("s") * _NC + lax.axis_index("c")
    base = wid * _BPW

    pltpu.sync_copy(fl_hbm, fl_v)

    zeros = jnp.zeros((_L,), jnp.int32)
    for j in range(_MAXU // _L):
        hist_v[pl.ds(j * _L, _L)] = zeros

    ones = jnp.ones((_L,), jnp.int32)

    def scan_step(i, carry):
        for j in range(8):
            v = fl_v[pl.ds((i * 8 + j) * _L, _L)]
            plsc.store_scatter(hist_v, [v], ones)
        return carry

    lax.fori_loop(0, _BATCH // (_L * 8), scan_step, jnp.int32(0))

    running = jnp.int32(0)
    for j in range(_MAXU // _L):
        seg = hist_v[pl.ds(j * _L, _L)]
        inc = plsc.cumsum(seg)
        rank_v[pl.ds(j * _L, _L)] = (inc - seg) + running
        running = running + jnp.sum(seg)

    for i in range(_BPW // _L):
        v = fl_v[pl.ds(base + i * _L, _L)]
        r = plsc.load_gather(rank_v, [v])
        idx2_v[i * _L // _IDX_CHUNK, pl.ds((i * _L) % _IDX_CHUNK, _L)] = r

    copies = []
    for g in range(_BPW // _IDX_CHUNK):
        copies.append(
            pltpu.async_copy(
                w_hbm.at[idx2_v.at[g]],
                rows_v.at[pl.ds(g * _IDX_CHUNK, _IDX_CHUNK)],
                sem,
            )
        )
    for c in copies:
        c.wait()

    pltpu.sync_copy(rows_v, out_hbm.at[pl.ds(base, _BPW)])


def kernel(file_list, W):
    mesh = plsc.VectorSubcoreMesh(
        core_axis_name="c", subcore_axis_name="s", num_cores=_NC, num_subcores=_NS
    )
    run = functools.partial(
        pl.kernel,
        out_type=jax.ShapeDtypeStruct((_BATCH, _DIM), jnp.float32),
        mesh=mesh,
        scratch_types=[
            pltpu.VMEM((_BATCH,), jnp.int32),
            pltpu.VMEM((_MAXU,), jnp.int32),
            pltpu.VMEM((_MAXU,), jnp.int32),
            pltpu.VMEM((_BPW // _IDX_CHUNK, _IDX_CHUNK), jnp.int32),
            pltpu.VMEM((_BPW, _DIM), jnp.float32),
            pltpu.SemaphoreType.DMA,
        ],
        compiler_params=pltpu.CompilerParams(needs_layout_passes=False),
    )(_body)
    return run(file_list, W)


# speedup vs baseline: 20.7016x; 1.3885x over previous
import functools

import jax
import jax.numpy as jnp
from jax import lax
from jax.experimental import pallas as pl
from jax.experimental.pallas import tpu as pltpu
from jax.experimental.pallas import tpu_sc as plsc

_DIM = 128
_MAXU = 64
_BATCH = 16384
_L = 16
_NC = 2
_NS = 16
_NW = _NC * _NS
_BPW = _BATCH // _NW
_IDX_CHUNK = 128


def _body(fl_hbm, w_hbm, out_hbm, fl_v, hist_v, rank_v, idx2_v, rows_v, w_sh, sem):
    sid = lax.axis_index("s")
    wid = sid * _NC + lax.axis_index("c")
    base = wid * _BPW

    @pl.when(sid == 0)
    def _():
        pltpu.sync_copy(w_hbm, w_sh)

    pltpu.sync_copy(fl_hbm, fl_v)

    zeros = jnp.zeros((_L,), jnp.int32)
    for j in range(_MAXU // _L):
        hist_v[pl.ds(j * _L, _L)] = zeros

    ones = jnp.ones((_L,), jnp.int32)

    def scan_step(i, carry):
        for j in range(8):
            v = fl_v[pl.ds((i * 8 + j) * _L, _L)]
            plsc.store_scatter(hist_v, [v], ones)
        return carry

    lax.fori_loop(0, _BATCH // (_L * 8), scan_step, jnp.int32(0))

    running = jnp.int32(0)
    for j in range(_MAXU // _L):
        seg = hist_v[pl.ds(j * _L, _L)]
        inc = plsc.cumsum(seg)
        rank_v[pl.ds(j * _L, _L)] = (inc - seg) + running
        running = running + jnp.sum(seg)

    for i in range(_BPW // _L):
        v = fl_v[pl.ds(base + i * _L, _L)]
        r = plsc.load_gather(rank_v, [v])
        idx2_v[i * _L // _IDX_CHUNK, pl.ds((i * _L) % _IDX_CHUNK, _L)] = r

    plsc.subcore_barrier()
    copies = []
    for g in range(_BPW // _IDX_CHUNK):
        copies.append(
            pltpu.async_copy(
                w_sh.at[idx2_v.at[g]],
                rows_v.at[pl.ds(g * _IDX_CHUNK, _IDX_CHUNK)],
                sem,
            )
        )
    for c in copies:
        c.wait()

    pltpu.sync_copy(rows_v, out_hbm.at[pl.ds(base, _BPW)])


def kernel(file_list, W):
    mesh = plsc.VectorSubcoreMesh(
        core_axis_name="c", subcore_axis_name="s", num_cores=_NC, num_subcores=_NS
    )
    run = functools.partial(
        pl.kernel,
        out_type=jax.ShapeDtypeStruct((_BATCH, _DIM), jnp.float32),
        mesh=mesh,
        scratch_types=[
            pltpu.VMEM((_BATCH,), jnp.int32),
            pltpu.VMEM((_MAXU,), jnp.int32),
            pltpu.VMEM((_MAXU,), jnp.int32),
            pltpu.VMEM((_BPW // _IDX_CHUNK, _IDX_CHUNK), jnp.int32),
            pltpu.VMEM((_BPW, _DIM), jnp.float32),
            pltpu.MemorySpace.VMEM_SHARED((_MAXU, _DIM), jnp.float32),
            pltpu.SemaphoreType.DMA,
        ],
        compiler_params=pltpu.CompilerParams(needs_layout_passes=False),
    )(_body)
    return run(file_list, W)


# speedup vs baseline: 25.1963x; 1.2171x over previous
import functools

import jax
import jax.numpy as jnp
from jax import lax
from jax.experimental import pallas as pl
from jax.experimental.pallas import tpu as pltpu
from jax.experimental.pallas import tpu_sc as plsc

_DIM = 128
_MAXU = 64
_BATCH = 16384
_L = 16
_NC = 2
_NS = 16
_NW = _NC * _NS
_BPW = _BATCH // _NW
_HPW = _BATCH // _NS
_IDX_CHUNK = 128


def _body(fl_hbm, w_hbm, out_hbm,
          flh_v, flm_v, hist_v, histall_v, rank_v, idx2_v, rows_v,
          w_sh, hist_sh, sem):
    sid = lax.axis_index("s")
    wid = sid * _NC + lax.axis_index("c")
    base = wid * _BPW

    @pl.when(sid == 0)
    def _():
        pltpu.sync_copy(w_hbm, w_sh)

    pltpu.sync_copy(fl_hbm.at[pl.ds(sid * _HPW, _HPW)], flh_v)
    pltpu.sync_copy(fl_hbm.at[pl.ds(base, _BPW)], flm_v)

    zeros = jnp.zeros((_L,), jnp.int32)
    for j in range(_MAXU // _L):
        hist_v[pl.ds(j * _L, _L)] = zeros

    ones = jnp.ones((_L,), jnp.int32)
    for i in range(_HPW // _L):
        v = flh_v[pl.ds(i * _L, _L)]
        plsc.store_scatter(hist_v, [v], ones)

    pltpu.sync_copy(hist_v, hist_sh.at[sid])
    plsc.subcore_barrier()
    pltpu.sync_copy(hist_sh, histall_v)

    running = jnp.int32(0)
    for j in range(_MAXU // _L):
        acc = zeros
        for t in range(_NS):
            acc = acc + histall_v[t, pl.ds(j * _L, _L)]
        pres = (acc > 0).astype(jnp.int32)
        inc = plsc.cumsum(pres)
        rank_v[pl.ds(j * _L, _L)] = (inc - pres) + running
        running = running + jnp.sum(pres)

    for i in range(_BPW // _L):
        v = flm_v[pl.ds(i * _L, _L)]
        r = plsc.load_gather(rank_v, [v])
        idx2_v[i * _L // _IDX_CHUNK, pl.ds((i * _L) % _IDX_CHUNK, _L)] = r

    copies = []
    for g in range(_BPW // _IDX_CHUNK):
        copies.append(
            pltpu.async_copy(
                w_sh.at[idx2_v.at[g]],
                rows_v.at[pl.ds(g * _IDX_CHUNK, _IDX_CHUNK)],
                sem,
            )
        )
    for c in copies:
        c.wait()

    pltpu.sync_copy(rows_v, out_hbm.at[pl.ds(base, _BPW)])


def kernel(file_list, W):
    mesh = plsc.VectorSubcoreMesh(
        core_axis_name="c", subcore_axis_name="s", num_cores=_NC, num_subcores=_NS
    )
    run = functools.partial(
        pl.kernel,
        out_type=jax.ShapeDtypeStruct((_BATCH, _DIM), jnp.float32),
        mesh=mesh,
        scratch_types=[
            pltpu.VMEM((_HPW,), jnp.int32),
            pltpu.VMEM((_BPW,), jnp.int32),
            pltpu.VMEM((_MAXU,), jnp.int32),
            pltpu.VMEM((_NS, _MAXU), jnp.int32),
            pltpu.VMEM((_MAXU,), jnp.int32),
            pltpu.VMEM((_BPW // _IDX_CHUNK, _IDX_CHUNK), jnp.int32),
            pltpu.VMEM((_BPW, _DIM), jnp.float32),
            pltpu.MemorySpace.VMEM_SHARED((_MAXU, _DIM), jnp.float32),
            pltpu.MemorySpace.VMEM_SHARED((_NS, _MAXU), jnp.int32),
            pltpu.SemaphoreType.DMA,
        ],
        compiler_params=pltpu.CompilerParams(needs_layout_passes=False),
    )(_body)
    return run(file_list, W)
